# Initial kernel scaffold; baseline (speedup 1.0000x reference)
import jax
import jax.numpy as jnp
from jax.experimental import pallas as pl


def kernel(in_feat, edge_index, W, b):
    raise NotImplementedError("write your pallas kernel here")


# speedup vs baseline: 4.6297x; 4.6297x over previous
import functools

import jax
import jax.numpy as jnp
from jax import lax
from jax.experimental import pallas as pl
from jax.experimental.pallas import tpu as pltpu
from jax.experimental.pallas import tpu_sc as plsc

N = 10000
E = 320000
D = 128

NC = 2
NS = 16
NW = NC * NS

C = 128
NCHUNK = E // C
NCHUNK2 = (2 * E) // C

HPAD = 20480

_ROWS_PER_SUB = N // NS
_HROWS_PER_SUB = HPAD // NS

_SC_PARAMS = pltpu.CompilerParams(use_tc_tiling_on_sc=False)


def _worker_id():
  return lax.axis_index("s") * NC + lax.axis_index("c")


def _chunk_range(wid, nchunk):
  base = nchunk // NW
  extra = nchunk % NW
  start = wid * base + jnp.minimum(wid, extra)
  count = base + jnp.where(wid < extra, 1, 0)
  return start, count


def _deg_body(idx_hbm, zeros_hbm, ones_hbm, out_hbm, idx_v, ones_v, hist):
  c = lax.axis_index("c")
  s = lax.axis_index("s")
  wid = _worker_id()

  pltpu.sync_copy(zeros_hbm.at[pl.ds(s * _HROWS_PER_SUB, _HROWS_PER_SUB)],
                  hist.at[pl.ds(s * _HROWS_PER_SUB, _HROWS_PER_SUB)])
  pltpu.sync_copy(ones_hbm, ones_v)
  plsc.subcore_barrier()

  start, count = _chunk_range(wid, NCHUNK2)

  def body(j, carry):
    chunk = start + j
    pltpu.sync_copy(idx_hbm.at[chunk], idx_v.at[0])
    pltpu.sync_copy(ones_v, hist.at[idx_v.at[0]], add=True)
    return carry

  lax.fori_loop(0, count, body, 0, unroll=False)
  plsc.subcore_barrier()

  pltpu.sync_copy(hist.at[pl.ds(s * _HROWS_PER_SUB, _HROWS_PER_SUB)],
                  out_hbm.at[c, pl.ds(s * _HROWS_PER_SUB, _HROWS_PER_SUB)])


_deg_kernel = functools.partial(
    pl.kernel,
    out_type=jax.ShapeDtypeStruct((NC, HPAD, 16), jnp.float32),
    mesh=plsc.VectorSubcoreMesh(core_axis_name="c", subcore_axis_name="s"),
    scratch_types=[
        pltpu.VMEM((1, C), jnp.int32),
        pltpu.VMEM((C, 16), jnp.float32),
        pltpu.VMEM_SHARED((HPAD, 16), jnp.float32),
    ],
    compiler_params=_SC_PARAMS,
)(_deg_body)


def _agg_body(h_hbm, src_hbm, dst_hbm, zeros_hbm, out_hbm,
              src_v, dst_v, rows_v, hist, sem):
  c = lax.axis_index("c")
  s = lax.axis_index("s")
  wid = _worker_id()

  pltpu.sync_copy(zeros_hbm.at[pl.ds(s * _ROWS_PER_SUB, _ROWS_PER_SUB)],
                  hist.at[pl.ds(s * _ROWS_PER_SUB, _ROWS_PER_SUB)])
  plsc.subcore_barrier()

  start, count = _chunk_range(wid, NCHUNK)

  def body(j, carry):
    chunk = start + j
    pltpu.sync_copy(src_hbm.at[chunk], src_v.at[0])
    pltpu.sync_copy(dst_hbm.at[chunk], dst_v.at[0])
    pltpu.async_copy(h_hbm.at[src_v.at[0]], rows_v, sem).wait()
    pltpu.sync_copy(rows_v, hist.at[dst_v.at[0]], add=True)
    return carry

  lax.fori_loop(0, count, body, 0, unroll=False)
  plsc.subcore_barrier()

  pltpu.sync_copy(hist.at[pl.ds(s * _ROWS_PER_SUB, _ROWS_PER_SUB)],
                  out_hbm.at[c, pl.ds(s * _ROWS_PER_SUB, _ROWS_PER_SUB)])


_agg_kernel = functools.partial(
    pl.kernel,
    out_type=jax.ShapeDtypeStruct((NC, N, D), jnp.float32),
    mesh=plsc.VectorSubcoreMesh(core_axis_name="c", subcore_axis_name="s"),
    scratch_types=[
        pltpu.VMEM((1, C), jnp.int32),
        pltpu.VMEM((1, C), jnp.int32),
        pltpu.VMEM((C, D), jnp.float32),
        pltpu.VMEM_SHARED((N, D), jnp.float32),
        pltpu.SemaphoreType.DMA,
    ],
    compiler_params=_SC_PARAMS,
)(_agg_body)


_RB = 2000


def _norm_scale_body(do0, do1, di0, di1, x, h_out, nd_out):
  deg_out = do0[0, :, 0] + do1[0, :, 0]
  deg_in = di0[0, :, 0] + di1[0, :, 0]
  norm_src = lax.rsqrt(jnp.maximum(deg_out, 1.0))
  norm_dst = lax.rsqrt(jnp.maximum(deg_in, 1.0))
  h_out[...] = x[...] * norm_src[:, None]
  nd_out[...] = norm_dst[:, None]


def _norm_scale(hist_parts, in_feat):
  nb = N // _RB
  return pl.pallas_call(
      _norm_scale_body,
      grid=(nb,),
      in_specs=[
          pl.BlockSpec((1, _RB, 16), lambda i: (0, i, 0)),
          pl.BlockSpec((1, _RB, 16), lambda i: (1, i, 0)),
          pl.BlockSpec((1, _RB, 16), lambda i: (0, i + nb, 0)),
          pl.BlockSpec((1, _RB, 16), lambda i: (1, i + nb, 0)),
          pl.BlockSpec((_RB, D), lambda i: (i, 0)),
      ],
      out_specs=[
          pl.BlockSpec((_RB, D), lambda i: (i, 0)),
          pl.BlockSpec((_RB, 1), lambda i: (i, 0)),
      ],
      out_shape=[
          jax.ShapeDtypeStruct((N, D), jnp.float32),
          jax.ShapeDtypeStruct((N, 1), jnp.float32),
      ],
  )(hist_parts, hist_parts, hist_parts, hist_parts, in_feat)


def _out_body(p0, p1, nd, w, bias, out):
  a = (p0[0] + p1[0]) * nd[...]
  y = jnp.dot(a, w[...], preferred_element_type=jnp.float32) + bias[...]
  out[...] = jnp.maximum(y, 0.0)


def _final(agg_parts, norm_dst, W, b2d):
  nb = N // _RB
  return pl.pallas_call(
      _out_body,
      grid=(nb,),
      in_specs=[
          pl.BlockSpec((1, _RB, D), lambda i: (0, i, 0)),
          pl.BlockSpec((1, _RB, D), lambda i: (1, i, 0)),
          pl.BlockSpec((_RB, 1), lambda i: (i, 0)),
          pl.BlockSpec((D, D), lambda i: (0, 0)),
          pl.BlockSpec((1, D), lambda i: (0, 0)),
      ],
      out_specs=pl.BlockSpec((_RB, D), lambda i: (i, 0)),
      out_shape=jax.ShapeDtypeStruct((N, D), jnp.float32),
  )(agg_parts, agg_parts, norm_dst, W, b2d)


@jax.jit
def kernel(in_feat, edge_index, W, b):
  src = edge_index[0]
  dst = edge_index[1]
  src2d = src.reshape(NCHUNK, C)
  dst2d = dst.reshape(NCHUNK, C)
  idx_all = jnp.concatenate([src, dst + N]).reshape(NCHUNK2, C)

  zeros_hist = jnp.zeros((HPAD, 16), jnp.float32)
  ones_c = jnp.zeros((C, 16), jnp.float32).at[:, 0].set(1.0)
  zeros_agg = jnp.zeros((N, D), jnp.float32)

  hist_parts = _deg_kernel(idx_all, zeros_hist, ones_c)
  h, norm_dst = _norm_scale(hist_parts, in_feat)
  agg_parts = _agg_kernel(h, src2d, dst2d, zeros_agg)
  return _final(agg_parts, norm_dst, W, b.reshape(1, D))
